# Initial kernel scaffold; baseline (speedup 1.0000x reference)
import jax
import jax.numpy as jnp
from jax.experimental import pallas as pl


def kernel(x, edges_1, edges_2, eps_noise, W1, b1, Wm, bm, Ws, bs):
    raise NotImplementedError("write your pallas kernel here")


# speedup vs baseline: 6.4607x; 6.4607x over previous
import functools

import jax
import jax.numpy as jnp
from jax import lax
from jax.experimental import pallas as pl
from jax.experimental.pallas import tpu as pltpu
from jax.experimental.pallas import tpu_sc as plsc

N = 10000
D = 128
H = 64
Z = 32
E1 = 160000
E2 = 160000

NC = 2
NS = 16
NW = NC * NS
K = 128
T1 = 40
EP = NW * T1 * K
NP = 10240
RPT = NP // NS
DW = 16

_f32 = jnp.float32


def _mesh():
    return plsc.VectorSubcoreMesh(core_axis_name="c", subcore_axis_name="s")


def _sc_deg(s1, d1, s2, d2):
    @functools.partial(
        pl.kernel,
        mesh=_mesh(),
        compiler_params=pltpu.CompilerParams(use_tc_tiling_on_sc=False),
        out_type=(
            jax.ShapeDtypeStruct((NC, NP, DW), _f32),
            jax.ShapeDtypeStruct((NC, NP, DW), _f32),
        ),
        scratch_types=[
            pltpu.VMEM((T1, K), jnp.int32),
            pltpu.VMEM((T1, K), jnp.int32),
            pltpu.VMEM((K, DW), _f32),
            pltpu.VMEM_SHARED((NP, DW), _f32),
            pltpu.VMEM_SHARED((NP, DW), _f32),
        ],
    )
    def k(s1h, d1h, s2h, d2h, dego_o, degi_o, sv, dv, onesv, degosh, degish):
        c = lax.axis_index("c")
        s = lax.axis_index("s")
        w = c * NS + s
        r0 = s * RPT

        def z16(i, _):
            onesv[i, pl.ds(0, 16)] = jnp.zeros((16,), _f32)
            return 0
        lax.fori_loop(0, K, z16, 0)

        for b in range(RPT // K):
            pltpu.sync_copy(onesv, degosh.at[pl.ds(r0 + b * K, K), :])
            pltpu.sync_copy(onesv, degish.at[pl.ds(r0 + b * K, K), :])

        def s16(i, _):
            onesv[i, pl.ds(0, 16)] = jnp.ones((16,), _f32)
            return 0
        lax.fori_loop(0, K, s16, 0)

        plsc.subcore_barrier()

        for sh, dh in ((s1h, d1h), (s2h, d2h)):
            pltpu.sync_copy(sh.at[pl.ds(w * T1, T1), :], sv)
            pltpu.sync_copy(dh.at[pl.ds(w * T1, T1), :], dv)

            def step(t, _):
                pltpu.sync_copy(onesv, degosh.at[sv.at[t]], add=True)
                pltpu.sync_copy(onesv, degish.at[dv.at[t]], add=True)
                return 0
            lax.fori_loop(0, T1, step, 0)

        plsc.subcore_barrier()
        pltpu.sync_copy(degosh.at[pl.ds(r0, RPT), :], dego_o.at[c, pl.ds(r0, RPT), :])
        pltpu.sync_copy(degish.at[pl.ds(r0, RPT), :], degi_o.at[c, pl.ds(r0, RPT), :])

    return k(s1, d1, s2, d2)


def _sc_phase1(r1, r2, s1, d1, s2, d2):
    @functools.partial(
        pl.kernel,
        mesh=_mesh(),
        compiler_params=pltpu.CompilerParams(use_tc_tiling_on_sc=False),
        out_type=jax.ShapeDtypeStruct((NC, NP, D), _f32),
        scratch_types=[
            pltpu.VMEM((T1, K), jnp.int32),
            pltpu.VMEM((T1, K), jnp.int32),
            pltpu.VMEM((K, D), _f32),
            pltpu.VMEM_SHARED((NP, D), _f32),
            pltpu.SemaphoreType.DMA,
        ],
    )
    def k(r1h, r2h, s1h, d1h, s2h, d2h, agg_o, sv, dv, rowsv, aggsh, sem):
        c = lax.axis_index("c")
        s = lax.axis_index("s")
        w = c * NS + s
        r0 = s * RPT

        def z128(i, _):
            rowsv[i // 8, pl.ds((i % 8) * 16, 16)] = jnp.zeros((16,), _f32)
            return 0
        lax.fori_loop(0, K * (D // 16), z128, 0)

        for b in range(RPT // K):
            pltpu.sync_copy(rowsv, aggsh.at[pl.ds(r0 + b * K, K), :])

        plsc.subcore_barrier()

        for tbl, sh, dh in ((r1h, s1h, d1h), (r2h, s2h, d2h)):
            pltpu.sync_copy(sh.at[pl.ds(w * T1, T1), :], sv)
            pltpu.sync_copy(dh.at[pl.ds(w * T1, T1), :], dv)

            def step(t, _):
                pltpu.async_copy(tbl.at[sv.at[t]], rowsv, sem).wait()
                pltpu.sync_copy(rowsv, aggsh.at[dv.at[t]], add=True)
                return 0
            lax.fori_loop(0, T1, step, 0)

        plsc.subcore_barrier()
        pltpu.sync_copy(aggsh.at[pl.ds(r0, RPT), :], agg_o.at[c, pl.ds(r0, RPT), :])

    return k(r1, r2, s1, d1, s2, d2)


def _sc_phase2(hn, sall, dall):
    T2 = 2 * T1

    @functools.partial(
        pl.kernel,
        mesh=_mesh(),
        compiler_params=pltpu.CompilerParams(use_tc_tiling_on_sc=False),
        out_type=jax.ShapeDtypeStruct((NC, NP, H), _f32),
        scratch_types=[
            pltpu.VMEM((T2, K), jnp.int32),
            pltpu.VMEM((T2, K), jnp.int32),
            pltpu.VMEM((K, H), _f32),
            pltpu.VMEM_SHARED((NP, H), _f32),
            pltpu.SemaphoreType.DMA,
        ],
    )
    def k(hnh, sh, dh, a_o, sv, dv, rowsv, accsh, sem):
        c = lax.axis_index("c")
        s = lax.axis_index("s")
        w = c * NS + s
        r0 = s * RPT

        def z64(i, _):
            rowsv[i // 4, pl.ds((i % 4) * 16, 16)] = jnp.zeros((16,), _f32)
            return 0
        lax.fori_loop(0, K * (H // 16), z64, 0)

        for b in range(RPT // K):
            pltpu.sync_copy(rowsv, accsh.at[pl.ds(r0 + b * K, K), :])

        plsc.subcore_barrier()

        pltpu.sync_copy(sh.at[pl.ds(w * T2, T2), :], sv)
        pltpu.sync_copy(dh.at[pl.ds(w * T2, T2), :], dv)

        def step(t, _):
            pltpu.async_copy(hnh.at[sv.at[t]], rowsv, sem).wait()
            pltpu.sync_copy(rowsv, accsh.at[dv.at[t]], add=True)
            return 0
        lax.fori_loop(0, T2, step, 0)

        plsc.subcore_barrier()
        pltpu.sync_copy(accsh.at[pl.ds(r0, RPT), :], a_o.at[c, pl.ds(r0, RPT), :])

    return k(hn, sall, dall)


_BR = 640


def _prep_body(x_ref, r1_ref, r2_ref):
    i = pl.program_id(0)
    row = i * _BR + lax.broadcasted_iota(jnp.int32, (_BR, 1), 0)
    mask = row < N
    xv = x_ref[...]
    r1_ref[...] = jnp.where(mask, jnp.maximum(xv + 1.0, 0.0), 0.0)
    r2_ref[...] = jnp.where(mask, jnp.maximum(xv + 2.0, 0.0), 0.0)


def _tc_prep(x_pad):
    return pl.pallas_call(
        _prep_body,
        grid=(NP // _BR,),
        in_specs=[pl.BlockSpec((_BR, D), lambda i: (i, 0))],
        out_specs=(pl.BlockSpec((_BR, D), lambda i: (i, 0)),
                   pl.BlockSpec((_BR, D), lambda i: (i, 0))),
        out_shape=(jax.ShapeDtypeStruct((NP, D), _f32),
                   jax.ShapeDtypeStruct((NP, D), _f32)),
    )(x_pad)


def _mid_body(x_ref, aggp_ref, degop_ref, w1_ref, b1_ref, hn_ref):
    i = pl.program_id(0)
    row = i * _BR + lax.broadcasted_iota(jnp.int32, (_BR, 1), 0)
    xa = x_ref[...] + aggp_ref[0] + aggp_ref[1]
    h = jnp.tanh(
        lax.dot_general(xa, w1_ref[...], (((1,), (0,)), ((), ())),
                        precision=lax.Precision.HIGHEST,
                        preferred_element_type=_f32)
        + b1_ref[...])
    dego = degop_ref[0, :, 0:1] + degop_ref[1, :, 0:1]
    hn = h * lax.rsqrt(jnp.maximum(dego, 1.0))
    hn_ref[...] = jnp.where(row < N, hn, 0.0)


def _tc_mid(x_pad, aggp, degop, W1, b1):
    return pl.pallas_call(
        _mid_body,
        grid=(NP // _BR,),
        in_specs=[
            pl.BlockSpec((_BR, D), lambda i: (i, 0)),
            pl.BlockSpec((NC, _BR, D), lambda i: (0, i, 0)),
            pl.BlockSpec((NC, _BR, DW), lambda i: (0, i, 0)),
            pl.BlockSpec((D, H), lambda i: (0, 0)),
            pl.BlockSpec((1, H), lambda i: (0, 0)),
        ],
        out_specs=pl.BlockSpec((_BR, H), lambda i: (i, 0)),
        out_shape=jax.ShapeDtypeStruct((NP, H), _f32),
    )(x_pad, aggp, degop, W1, b1)


_BRO = 400


def _out_body(ap_ref, degip_ref, eps_ref, wm_ref, ws_ref, bm_ref, bs_ref,
              z_ref, m_ref, s_ref):
    degi = degip_ref[0, :, 0:1] + degip_ref[1, :, 0:1]
    a = (ap_ref[0] + ap_ref[1]) * lax.rsqrt(jnp.maximum(degi, 1.0))
    m = lax.dot_general(a, wm_ref[...], (((1,), (0,)), ((), ())),
                        precision=lax.Precision.HIGHEST,
                        preferred_element_type=_f32) + bm_ref[...]
    sd = jnp.maximum(
        lax.dot_general(a, ws_ref[...], (((1,), (0,)), ((), ())),
                        precision=lax.Precision.HIGHEST,
                        preferred_element_type=_f32) + bs_ref[...], 0.0) + 0.0001
    z_ref[...] = eps_ref[...] * sd + m
    m_ref[...] = m
    s_ref[...] = sd


def _tc_out(ap, degip, eps, Wm, Ws, bm, bs):
    return pl.pallas_call(
        _out_body,
        grid=(N // _BRO,),
        in_specs=[
            pl.BlockSpec((NC, _BRO, H), lambda i: (0, i, 0)),
            pl.BlockSpec((NC, _BRO, DW), lambda i: (0, i, 0)),
            pl.BlockSpec((_BRO, Z), lambda i: (i, 0)),
            pl.BlockSpec((H, Z), lambda i: (0, 0)),
            pl.BlockSpec((H, Z), lambda i: (0, 0)),
            pl.BlockSpec((1, Z), lambda i: (0, 0)),
            pl.BlockSpec((1, Z), lambda i: (0, 0)),
        ],
        out_specs=(pl.BlockSpec((_BRO, Z), lambda i: (i, 0)),
                   pl.BlockSpec((_BRO, Z), lambda i: (i, 0)),
                   pl.BlockSpec((_BRO, Z), lambda i: (i, 0))),
        out_shape=(jax.ShapeDtypeStruct((N, Z), _f32),
                   jax.ShapeDtypeStruct((N, Z), _f32),
                   jax.ShapeDtypeStruct((N, Z), _f32)),
    )(ap, degip, eps, Wm, Ws, bm, bs)


def _pad_edges(idx, ep, fill):
    pad = jnp.full((ep - idx.shape[0],), fill, jnp.int32)
    return jnp.concatenate([idx, pad]).reshape(ep // K, K)


def kernel(x, edges_1, edges_2, eps_noise, W1, b1, Wm, bm, Ws, bs):
    x_pad = jnp.pad(x, ((0, NP - N), (0, 0)))
    r1, r2 = _tc_prep(x_pad)

    s1 = _pad_edges(edges_1[0], EP, N)
    d1 = _pad_edges(edges_1[1], EP, N)
    s2 = _pad_edges(edges_2[0], EP, N)
    d2 = _pad_edges(edges_2[1], EP, N)

    degop, degip = _sc_deg(s1, d1, s2, d2)
    aggp = _sc_phase1(r1, r2, s1, d1, s2, d2)

    hn = _tc_mid(x_pad, aggp, degop, W1, b1.reshape(1, H))

    sall = jnp.concatenate([s1, s2], axis=0)
    dall = jnp.concatenate([d1, d2], axis=0)
    ap = _sc_phase2(hn, sall, dall)

    z, m, sd = _tc_out(ap, degip, eps_noise, Wm, Ws,
                       bm.reshape(1, Z), bs.reshape(1, Z))
    return (z, m, sd)


# speedup vs baseline: 7.3162x; 1.1324x over previous
import functools

import jax
import jax.numpy as jnp
from jax import lax
from jax.experimental import pallas as pl
from jax.experimental.pallas import tpu as pltpu
from jax.experimental.pallas import tpu_sc as plsc

N = 10000
D = 128
H = 64
Z = 32
E1 = 160000
E2 = 160000

NC = 2
NS = 16
NW = NC * NS
K = 128
T1 = 40
EP = NW * T1 * K
NP = 10240
RPT = NP // NS
DW = 16
CI = 1
CR = CI * K
NCH = T1 // CI

_f32 = jnp.float32


def _mesh():
    return plsc.VectorSubcoreMesh(core_axis_name="c", subcore_axis_name="s")


def _sc_deg(s1, d1, s2, d2):
    @functools.partial(
        pl.kernel,
        mesh=_mesh(),
        compiler_params=pltpu.CompilerParams(use_tc_tiling_on_sc=False),
        out_type=(
            jax.ShapeDtypeStruct((NC, NP, DW), _f32),
            jax.ShapeDtypeStruct((NC, NP, DW), _f32),
        ),
        scratch_types=[
            pltpu.VMEM((T1, K), jnp.int32),
            pltpu.VMEM((T1, K), jnp.int32),
            pltpu.VMEM((K, DW), _f32),
            pltpu.VMEM_SHARED((NP, DW), _f32),
            pltpu.VMEM_SHARED((NP, DW), _f32),
        ],
    )
    def k(s1h, d1h, s2h, d2h, dego_o, degi_o, sv, dv, onesv, degosh, degish):
        c = lax.axis_index("c")
        s = lax.axis_index("s")
        w = c * NS + s
        r0 = s * RPT

        def z16(i, _):
            onesv[i, pl.ds(0, 16)] = jnp.zeros((16,), _f32)
            return 0
        lax.fori_loop(0, K, z16, 0)

        for b in range(RPT // K):
            pltpu.sync_copy(onesv, degosh.at[pl.ds(r0 + b * K, K), :])
            pltpu.sync_copy(onesv, degish.at[pl.ds(r0 + b * K, K), :])

        def s16(i, _):
            onesv[i, pl.ds(0, 16)] = jnp.ones((16,), _f32)
            return 0
        lax.fori_loop(0, K, s16, 0)

        plsc.subcore_barrier()

        for sh, dh in ((s1h, d1h), (s2h, d2h)):
            pltpu.sync_copy(sh.at[pl.ds(w * T1, T1), :], sv)
            pltpu.sync_copy(dh.at[pl.ds(w * T1, T1), :], dv)

            def step(t, _):
                pltpu.sync_copy(onesv, degosh.at[sv.at[t]], add=True)
                pltpu.sync_copy(onesv, degish.at[dv.at[t]], add=True)
                return 0
            lax.fori_loop(0, T1, step, 0)

        plsc.subcore_barrier()
        pltpu.sync_copy(degosh.at[pl.ds(r0, RPT), :], dego_o.at[c, pl.ds(r0, RPT), :])
        pltpu.sync_copy(degish.at[pl.ds(r0, RPT), :], degi_o.at[c, pl.ds(r0, RPT), :])

    return k(s1, d1, s2, d2)


def _sc_phase1(r1, r2, s1, d1, s2, d2):
    @functools.partial(
        pl.kernel,
        mesh=_mesh(),
        compiler_params=pltpu.CompilerParams(use_tc_tiling_on_sc=False),
        out_type=jax.ShapeDtypeStruct((NC, NP, D), _f32),
        scratch_types=[
            pltpu.VMEM((T1, K), jnp.int32),
            pltpu.VMEM((T1, K), jnp.int32),
            pltpu.VMEM((CR, D), _f32),
            pltpu.VMEM((CR, D), _f32),
            pltpu.VMEM_SHARED((NP, D), _f32),
            pltpu.SemaphoreType.DMA,
            pltpu.SemaphoreType.DMA,
        ],
    )
    def k(r1h, r2h, s1h, d1h, s2h, d2h, agg_o, sv, dv,
          rows0, rows1, aggsh, semg0, semg1):
        c = lax.axis_index("c")
        s = lax.axis_index("s")
        w = c * NS + s
        r0 = s * RPT

        def z128(i, _):
            rows0[i // 8, pl.ds((i % 8) * 16, 16)] = jnp.zeros((16,), _f32)
            return 0
        lax.fori_loop(0, K * (D // 16), z128, 0)

        for b in range(RPT // K):
            pltpu.sync_copy(rows0.at[pl.ds(0, K), :],
                            aggsh.at[pl.ds(r0 + b * K, K), :])

        plsc.subcore_barrier()

        bufs = (rows0, rows1)
        semg = (semg0, semg1)

        for tbl, sh, dh in ((r1h, s1h, d1h), (r2h, s2h, d2h)):
            pltpu.sync_copy(sh.at[pl.ds(w * T1, T1), :], sv)
            pltpu.sync_copy(dh.at[pl.ds(w * T1, T1), :], dv)

            for b in range(2):
                pltpu.async_copy(tbl.at[sv.at[b]],
                                 bufs[b], semg[b])

            def step2(t2, _):
                for b in range(2):
                    t = 2 * t2 + b
                    pltpu.make_async_copy(tbl.at[sv.at[0]],
                                          bufs[b], semg[b]).wait()
                    pltpu.sync_copy(bufs[b],
                                    aggsh.at[dv.at[t]],
                                    add=True)

                    @pl.when(t + 2 < NCH)
                    def _():
                        pltpu.async_copy(
                            tbl.at[sv.at[t + 2]],
                            bufs[b], semg[b])
                return 0
            lax.fori_loop(0, NCH // 2, step2, 0)

        plsc.subcore_barrier()
        pltpu.sync_copy(aggsh.at[pl.ds(r0, RPT), :], agg_o.at[c, pl.ds(r0, RPT), :])

    return k(r1, r2, s1, d1, s2, d2)


def _sc_phase2(hn, sall, dall):
    T2 = 2 * T1

    @functools.partial(
        pl.kernel,
        mesh=_mesh(),
        compiler_params=pltpu.CompilerParams(use_tc_tiling_on_sc=False),
        out_type=jax.ShapeDtypeStruct((NC, NP, H), _f32),
        scratch_types=[
            pltpu.VMEM((T2, K), jnp.int32),
            pltpu.VMEM((T2, K), jnp.int32),
            pltpu.VMEM((CR, H), _f32),
            pltpu.VMEM((CR, H), _f32),
            pltpu.VMEM_SHARED((NP, H), _f32),
            pltpu.SemaphoreType.DMA,
            pltpu.SemaphoreType.DMA,
        ],
    )
    def k(hnh, sh, dh, a_o, sv, dv, rows0, rows1, accsh, semg0, semg1):
        c = lax.axis_index("c")
        s = lax.axis_index("s")
        w = c * NS + s
        r0 = s * RPT
        nch2 = T2 // CI

        def z64(i, _):
            rows0[i // 4, pl.ds((i % 4) * 16, 16)] = jnp.zeros((16,), _f32)
            return 0
        lax.fori_loop(0, K * (H // 16), z64, 0)

        for b in range(RPT // K):
            pltpu.sync_copy(rows0.at[pl.ds(0, K), :],
                            accsh.at[pl.ds(r0 + b * K, K), :])

        plsc.subcore_barrier()

        bufs = (rows0, rows1)
        semg = (semg0, semg1)

        pltpu.sync_copy(sh.at[pl.ds(w * T2, T2), :], sv)
        pltpu.sync_copy(dh.at[pl.ds(w * T2, T2), :], dv)

        for b in range(2):
            pltpu.async_copy(hnh.at[sv.at[b]],
                             bufs[b], semg[b])

        def step2(t2, _):
            for b in range(2):
                t = 2 * t2 + b
                pltpu.make_async_copy(hnh.at[sv.at[0]],
                                      bufs[b], semg[b]).wait()
                pltpu.sync_copy(bufs[b],
                                accsh.at[dv.at[t]],
                                add=True)

                @pl.when(t + 2 < nch2)
                def _():
                    pltpu.async_copy(
                        hnh.at[sv.at[t + 2]],
                        bufs[b], semg[b])
            return 0
        lax.fori_loop(0, nch2 // 2, step2, 0)

        plsc.subcore_barrier()
        pltpu.sync_copy(accsh.at[pl.ds(r0, RPT), :], a_o.at[c, pl.ds(r0, RPT), :])

    return k(hn, sall, dall)


_BR = 640


def _prep_body(x_ref, r1_ref, r2_ref):
    i = pl.program_id(0)
    row = i * _BR + lax.broadcasted_iota(jnp.int32, (_BR, 1), 0)
    mask = row < N
    xv = x_ref[...]
    r1_ref[...] = jnp.where(mask, jnp.maximum(xv + 1.0, 0.0), 0.0)
    r2_ref[...] = jnp.where(mask, jnp.maximum(xv + 2.0, 0.0), 0.0)


def _tc_prep(x_pad):
    return pl.pallas_call(
        _prep_body,
        grid=(NP // _BR,),
        in_specs=[pl.BlockSpec((_BR, D), lambda i: (i, 0))],
        out_specs=(pl.BlockSpec((_BR, D), lambda i: (i, 0)),
                   pl.BlockSpec((_BR, D), lambda i: (i, 0))),
        out_shape=(jax.ShapeDtypeStruct((NP, D), _f32),
                   jax.ShapeDtypeStruct((NP, D), _f32)),
    )(x_pad)


def _mid_body(x_ref, aggp_ref, degop_ref, w1_ref, b1_ref, hn_ref):
    i = pl.program_id(0)
    row = i * _BR + lax.broadcasted_iota(jnp.int32, (_BR, 1), 0)
    xa = x_ref[...] + aggp_ref[0] + aggp_ref[1]
    h = jnp.tanh(
        lax.dot_general(xa, w1_ref[...], (((1,), (0,)), ((), ())),
                        precision=lax.Precision.HIGHEST,
                        preferred_element_type=_f32)
        + b1_ref[...])
    dego = degop_ref[0, :, 0:1] + degop_ref[1, :, 0:1]
    hn = h * lax.rsqrt(jnp.maximum(dego, 1.0))
    hn_ref[...] = jnp.where(row < N, hn, 0.0)


def _tc_mid(x_pad, aggp, degop, W1, b1):
    return pl.pallas_call(
        _mid_body,
        grid=(NP // _BR,),
        in_specs=[
            pl.BlockSpec((_BR, D), lambda i: (i, 0)),
            pl.BlockSpec((NC, _BR, D), lambda i: (0, i, 0)),
            pl.BlockSpec((NC, _BR, DW), lambda i: (0, i, 0)),
            pl.BlockSpec((D, H), lambda i: (0, 0)),
            pl.BlockSpec((1, H), lambda i: (0, 0)),
        ],
        out_specs=pl.BlockSpec((_BR, H), lambda i: (i, 0)),
        out_shape=jax.ShapeDtypeStruct((NP, H), _f32),
    )(x_pad, aggp, degop, W1, b1)


_BRO = 400


def _out_body(ap_ref, degip_ref, eps_ref, wm_ref, ws_ref, bm_ref, bs_ref,
              z_ref, m_ref, s_ref):
    degi = degip_ref[0, :, 0:1] + degip_ref[1, :, 0:1]
    a = (ap_ref[0] + ap_ref[1]) * lax.rsqrt(jnp.maximum(degi, 1.0))
    m = lax.dot_general(a, wm_ref[...], (((1,), (0,)), ((), ())),
                        precision=lax.Precision.HIGHEST,
                        preferred_element_type=_f32) + bm_ref[...]
    sd = jnp.maximum(
        lax.dot_general(a, ws_ref[...], (((1,), (0,)), ((), ())),
                        precision=lax.Precision.HIGHEST,
                        preferred_element_type=_f32) + bs_ref[...], 0.0) + 0.0001
    z_ref[...] = eps_ref[...] * sd + m
    m_ref[...] = m
    s_ref[...] = sd


def _tc_out(ap, degip, eps, Wm, Ws, bm, bs):
    return pl.pallas_call(
        _out_body,
        grid=(N // _BRO,),
        in_specs=[
            pl.BlockSpec((NC, _BRO, H), lambda i: (0, i, 0)),
            pl.BlockSpec((NC, _BRO, DW), lambda i: (0, i, 0)),
            pl.BlockSpec((_BRO, Z), lambda i: (i, 0)),
            pl.BlockSpec((H, Z), lambda i: (0, 0)),
            pl.BlockSpec((H, Z), lambda i: (0, 0)),
            pl.BlockSpec((1, Z), lambda i: (0, 0)),
            pl.BlockSpec((1, Z), lambda i: (0, 0)),
        ],
        out_specs=(pl.BlockSpec((_BRO, Z), lambda i: (i, 0)),
                   pl.BlockSpec((_BRO, Z), lambda i: (i, 0)),
                   pl.BlockSpec((_BRO, Z), lambda i: (i, 0))),
        out_shape=(jax.ShapeDtypeStruct((N, Z), _f32),
                   jax.ShapeDtypeStruct((N, Z), _f32),
                   jax.ShapeDtypeStruct((N, Z), _f32)),
    )(ap, degip, eps, Wm, Ws, bm, bs)


def _pad_edges(idx, ep, fill):
    pad = jnp.full((ep - idx.shape[0],), fill, jnp.int32)
    return jnp.concatenate([idx, pad]).reshape(ep // K, K)


def kernel(x, edges_1, edges_2, eps_noise, W1, b1, Wm, bm, Ws, bs):
    x_pad = jnp.pad(x, ((0, NP - N), (0, 0)))
    r1, r2 = _tc_prep(x_pad)

    s1 = _pad_edges(edges_1[0], EP, N)
    d1 = _pad_edges(edges_1[1], EP, N)
    s2 = _pad_edges(edges_2[0], EP, N)
    d2 = _pad_edges(edges_2[1], EP, N)

    degop, degip = _sc_deg(s1, d1, s2, d2)
    aggp = _sc_phase1(r1, r2, s1, d1, s2, d2)

    hn = _tc_mid(x_pad, aggp, degop, W1, b1.reshape(1, H))

    sall = jnp.concatenate([s1, s2], axis=0)
    dall = jnp.concatenate([d1, d2], axis=0)
    ap = _sc_phase2(hn, sall, dall)

    z, m, sd = _tc_out(ap, degip, eps_noise, Wm, Ws,
                       bm.reshape(1, Z), bs.reshape(1, Z))
    return (z, m, sd)
